# baseline (speedup 1.0000x reference)
import jax
import jax.numpy as jnp
import numpy as np
from jax.experimental import pallas as pl
from jax.experimental.pallas import tpu as pltpu

HID = 64
N = 400
NN = N * N
B = 4
T = 32
NLAYERS = 2
NCLS = 2

_qf = np.float32(70.0) / np.float32(100.0)
_qs = _qf * np.float32(NN - 1)
_RANK_LO = int(np.floor(_qs)) + 1
_W_HI = np.float32(_qs - np.floor(_qs))
_W_LO = np.float32(np.float32(1.0) - _W_HI)

_INT_MIN = np.int32(-2147483648)
_INT_MAX = np.int32(2147483647)
_MAG = np.int32(0x7FFFFFFF)


def _sortable(bits):
    return jnp.where(bits < 0, bits ^ _MAG, bits)


_SPAN = np.int32(1 << 19)


def _percentile_blocks(aa, prev_ref, full_ref, step):
    m = _sortable(jax.lax.bitcast_convert_type(aa, jnp.int32))
    k = _RANK_LO
    red = lambda x: jnp.sum(x.astype(jnp.int32), axis=(1, 2), keepdims=True)

    def body(_, carry):
        res, bit = carry
        trial = res + bit
        c = red(m < trial)
        res = jnp.where(c < k, trial, res)
        return res, bit >> 1

    center = prev_ref[...]
    c1 = center - _SPAN
    n1 = red(m < c1)
    n2 = red(m < center + _SPAN)
    ok = ((step > 0) & jnp.all(n1 < k) & jnp.all(n2 >= k))
    short, _ = jax.lax.fori_loop(0, 20, body, (c1, _SPAN))

    @pl.when(jnp.logical_not(ok))
    def _full_search():
        neg = red(m < 0)
        res0 = jnp.where(neg >= k, _INT_MIN, np.int32(0))
        full, _ = jax.lax.fori_loop(0, 31, body, (res0, np.int32(1 << 30)))
        full_ref[...] = full

    m_lo = jnp.where(ok, short, full_ref[...])
    prev_ref[...] = m_lo
    cnt_le = red(m <= m_lo)
    m_hi_next = jnp.min(jnp.where(m > m_lo, m, _INT_MAX), axis=(1, 2),
                        keepdims=True)
    m_hi = jnp.where(cnt_le >= k + 1, m_lo, m_hi_next)
    f_lo = jax.lax.bitcast_convert_type(_sortable(m_lo), jnp.float32)
    f_hi = jax.lax.bitcast_convert_type(_sortable(m_hi), jnp.float32)
    return f_lo * _W_LO + f_hi * _W_HI


def _dot(x, y):
    return jax.lax.dot_general(x, y, (((1,), (0,)), ((), ())),
                               preferred_element_type=jnp.float32)


BPS = 8
HBPS = 16


def _h3_body(v_ref, wiT_ref, bi_ref, h3_ref):
    h3_ref[...] = (_dot(v_ref[...].reshape(HBPS * N, N), wiT_ref[...])
                   + bi_ref[...]).reshape(HBPS, N, HID)


def _stage1_body(h3_ref, a_ref, eps_ref,
                 A1_ref, c1_ref, A2_ref, c2_ref,
                 Ae_ref, ce_ref, WaT_ref, ba_ref,
                 hr_ref, ga_ref, prev_ref, full_ref):
    aa = a_ref[...]
    pct = _percentile_blocks(aa, prev_ref, full_ref,
                             pl.program_id(0))
    h3 = h3_ref[...]
    mask = (aa > pct).astype(jnp.float32)
    M = jax.lax.dot_general(mask, h3, (((2,), (1,)), ((0,), (0,))),
                            preferred_element_type=jnp.float32)
    for l in range(NLAYERS):
        x = (M + eps_ref[l] * h3).reshape(BPS * N, HID)
        x = jax.nn.relu(_dot(x, A1_ref[l]) + c1_ref[l])
        x2 = jax.nn.relu(_dot(x, A2_ref[l]) + c2_ref[l]).reshape(BPS, N, HID)
        xr = jnp.mean(x2, axis=1)
        pe = _dot(xr, Ae_ref[l]) + ce_ref[l]
        xe = pe * (jax.lax.erf(pe / np.float32(np.sqrt(2.0))) + 1) / 2
        ga = jax.nn.sigmoid(_dot(xe, WaT_ref[l]) + ba_ref[l])
        h_read = jnp.sum(x2 * ga[:, :, None], axis=1) * np.float32(1.0 / N)
        ga_ref[:, l, :] = ga
        hr_ref[:, l, :] = h_read


def _ln(x, g, b):
    m = x.mean(-1, keepdims=True)
    va = ((x - m) ** 2).mean(-1, keepdims=True)
    return g * (x - m) / jnp.sqrt(va + 1e-5) + b


def _bdot(x, y):
    return jax.lax.dot_general(x, y, (((2,), (1,)), ((0,), (0,))),
                               preferred_element_type=jnp.float32)


def _stage2_body(hr_ref, bias_ref,
                 WqT_ref, bq_ref, WkT_ref, bk_ref, WvT_ref, bv_ref,
                 WoT_ref, bo_ref, ln1g_ref, ln1b_ref,
                 W1T_ref, b1_ref, W2T_ref, b2_ref, ln2g_ref, ln2b_ref,
                 WLT_ref, bL_ref,
                 logit_ref, ta_ref, fG_ref, fT_ref, fL_ref):
    LB = NLAYERS * B
    hr4 = hr_ref[...]
    fG_ref[...] = jnp.mean(hr4, axis=2)
    hr2 = hr4.reshape(NLAYERS, B * T, HID)
    q = (_bdot(hr2, WqT_ref[...]) + bq_ref[...]).reshape(LB * T, HID)
    k = (_bdot(hr2, WkT_ref[...]) + bk_ref[...]).reshape(LB * T, HID)
    vv = (_bdot(hr2, WvT_ref[...]) + bv_ref[...]).reshape(LB * T, HID)
    sc = _dot(q, k.T) / np.float32(np.sqrt(float(HID))) + bias_ref[...]
    w = jax.nn.softmax(sc, axis=-1)
    for lb in range(LB):
        ta_ref[lb // B, lb % B, :, :] = \
            w[lb * T:(lb + 1) * T, lb * T:(lb + 1) * T]
    o = _dot(w, vv).reshape(NLAYERS, B * T, HID)
    o = _bdot(o, WoT_ref[...]) + bo_ref[...]
    xa = _ln(o, ln1g_ref[...], ln1b_ref[...])
    x2 = _bdot(jax.nn.relu(_bdot(xa, W1T_ref[...]) + b1_ref[...]),
               W2T_ref[...]) + b2_ref[...]
    xa = _ln(xa + x2, ln2g_ref[...], ln2b_ref[...])
    featT = jnp.sum(xa.reshape(NLAYERS, B, T, HID), axis=2)
    featL = _bdot(featT, WLT_ref[...]) + bL_ref[...]
    fT_ref[...] = featT
    fL_ref[...] = featL
    logit_ref[...] = jnp.sum(featL, axis=0)


def _bn_fold(p):
    s = p['g'] / jnp.sqrt(p['v'] + 1e-5)
    return s, p['b'] - p['m'] * s


@jax.jit
def kernel(v, a, t, sampling_endpoints, params):
    del t, sampling_endpoints
    layers = params['layers']
    v3 = v.reshape(B * T, N, N)
    a3 = a.reshape(B * T, N, N)

    wiT = params['init_W'].T
    bi = params['init_b'].reshape(1, HID)
    eps = jnp.stack([p['eps'] for p in layers])

    def fold_lin(W, bvec, bn):
        s, sh = _bn_fold(bn)
        return W.T * s[None, :], (bvec * s + sh).reshape(1, -1)

    A1, c1, A2, c2, Ae, ce, WaT, ba = [], [], [], [], [], [], [], []
    for p in layers:
        w, c = fold_lin(p['g_W1'], p['g_b1'], p['g_bn1']); A1.append(w); c1.append(c)
        w, c = fold_lin(p['g_W2'], p['g_b2'], p['g_bn2']); A2.append(w); c2.append(c)
        w, c = fold_lin(p['s_We'], p['s_be'], p['s_bn']); Ae.append(w); ce.append(c)
        WaT.append(p['s_Wa'].T); ba.append(p['s_ba'].reshape(1, N))
    st = jnp.stack

    def full(shape):
        return pl.BlockSpec(shape, lambda i: (0,) * len(shape))

    def blk(shape):
        return pl.BlockSpec(shape, lambda i: (i,) + (0,) * (len(shape) - 1))

    h3_all = pl.pallas_call(
        _h3_body,
        grid=(B * T // HBPS,),
        in_specs=[blk((HBPS, N, N)), full((N, HID)), full((1, HID))],
        out_specs=blk((HBPS, N, HID)),
        out_shape=jax.ShapeDtypeStruct((B * T, N, HID), jnp.float32),
    )(v3, wiT, bi)

    hr_all, ga_all = pl.pallas_call(
        _stage1_body,
        grid=(B * T // BPS,),
        in_specs=[
            blk((BPS, N, HID)), blk((BPS, N, N)),
            full((NLAYERS, 1, 1)),
            full((NLAYERS, HID, HID)), full((NLAYERS, 1, HID)),
            full((NLAYERS, HID, HID)), full((NLAYERS, 1, HID)),
            full((NLAYERS, HID, HID)), full((NLAYERS, 1, HID)),
            full((NLAYERS, HID, N)), full((NLAYERS, 1, N)),
        ],
        out_specs=[blk((BPS, NLAYERS, HID)), blk((BPS, NLAYERS, N))],
        out_shape=[
            jax.ShapeDtypeStruct((B * T, NLAYERS, HID), jnp.float32),
            jax.ShapeDtypeStruct((B * T, NLAYERS, N), jnp.float32),
        ],
        scratch_shapes=[pltpu.VMEM((BPS, 1, 1), jnp.int32),
                        pltpu.VMEM((BPS, 1, 1), jnp.int32)],
    )(h3_all, a3, eps, st(A1), st(c1), st(A2), st(c2),
      st(Ae), st(ce), st(WaT), st(ba))

    hr = hr_all.reshape(B, T, NLAYERS, HID).transpose(2, 0, 1, 3)
    node_att = ga_all.reshape(B, T, NLAYERS, N).transpose(0, 2, 1, 3)

    Wq, bq, Wk, bk, Wv, bv = [], [], [], [], [], []
    Wo, bo, g1, b1g, W1m, b1m, W2m, b2m, g2, b2g, WL, bL = ([] for _ in range(12))
    for p in layers:
        q3, k3, v3s = jnp.split(p['t_Win'], 3, axis=0)
        q3b, k3b, v3b = jnp.split(p['t_bin'], 3)
        Wq.append(q3.T); Wk.append(k3.T); Wv.append(v3s.T)
        bq.append(q3b.reshape(1, HID)); bk.append(k3b.reshape(1, HID))
        bv.append(v3b.reshape(1, HID))
        Wo.append(p['t_Wo'].T); bo.append(p['t_bo'].reshape(1, HID))
        g1.append(p['t_ln1g'].reshape(1, HID)); b1g.append(p['t_ln1b'].reshape(1, HID))
        W1m.append(p['t_W1'].T); b1m.append(p['t_b1'].reshape(1, 2 * HID))
        W2m.append(p['t_W2'].T); b2m.append(p['t_b2'].reshape(1, HID))
        g2.append(p['t_ln2g'].reshape(1, HID)); b2g.append(p['t_ln2b'].reshape(1, HID))
        WL.append(p['L_W'].T); bL.append(p['L_b'].reshape(1, NCLS))

    s2_ins = [st(Wq), st(bq), st(Wk), st(bk), st(Wv), st(bv),
              st(Wo), st(bo), st(g1), st(b1g),
              st(W1m), st(b1m), st(W2m), st(b2m), st(g2), st(b2g),
              st(WL), st(bL)]

    lbt = NLAYERS * B * T
    blkid = jnp.arange(lbt, dtype=jnp.int32) // T
    bias = jnp.where(blkid[:, None] == blkid[None, :],
                     jnp.float32(0), -jnp.inf)

    outs = pl.pallas_call(
        _stage2_body,
        grid=(1,),
        in_specs=[full((NLAYERS, B, T, HID)), full((lbt, lbt))]
                 + [full(x.shape) for x in s2_ins],
        out_specs=[full((B, NCLS)), full((NLAYERS, B, T, T)),
                   full((NLAYERS, B, HID)), full((NLAYERS, B, HID)),
                   full((NLAYERS, B, NCLS))],
        out_shape=[
            jax.ShapeDtypeStruct((B, NCLS), jnp.float32),
            jax.ShapeDtypeStruct((NLAYERS, B, T, T), jnp.float32),
            jax.ShapeDtypeStruct((NLAYERS, B, HID), jnp.float32),
            jax.ShapeDtypeStruct((NLAYERS, B, HID), jnp.float32),
            jax.ShapeDtypeStruct((NLAYERS, B, NCLS), jnp.float32),
        ],
    )(hr, bias, *s2_ins)

    logit, ta, fG, fT, fL = outs
    return (logit,
            node_att,
            ta.transpose(1, 0, 2, 3),
            fG.transpose(1, 0, 2),
            fT.transpose(1, 0, 2),
            fL.transpose(1, 0, 2))


# speedup vs baseline: 1.0384x; 1.0384x over previous
import jax
import jax.numpy as jnp
import numpy as np
from jax.experimental import pallas as pl
from jax.experimental.pallas import tpu as pltpu

HID = 64
N = 400
NN = N * N
B = 4
T = 32
NLAYERS = 2
NCLS = 2

_qf = np.float32(70.0) / np.float32(100.0)
_qs = _qf * np.float32(NN - 1)
_RANK_LO = int(np.floor(_qs)) + 1
_W_HI = np.float32(_qs - np.floor(_qs))
_W_LO = np.float32(np.float32(1.0) - _W_HI)

_INT_MIN = np.int32(-2147483648)
_INT_MAX = np.int32(2147483647)
_MAG = np.int32(0x7FFFFFFF)


def _sortable(bits):
    return jnp.where(bits < 0, bits ^ _MAG, bits)


_SPAN = np.int32(1 << 19)


def _percentile_blocks(aa, prev_ref, full_ref, step):
    m = _sortable(jax.lax.bitcast_convert_type(aa, jnp.int32))
    k = _RANK_LO
    red = lambda x: jnp.sum(x.astype(jnp.int32), axis=(1, 2), keepdims=True)

    def body(_, carry):
        res, bit = carry
        trial = res + bit
        c = red(m < trial)
        res = jnp.where(c < k, trial, res)
        return res, bit >> 1

    center = prev_ref[...]
    c1 = center - _SPAN
    n1 = red(m < c1)
    n2 = red(m < center + _SPAN)
    ok = ((step > 0) & jnp.all(n1 < k) & jnp.all(n2 >= k))
    short, _ = jax.lax.fori_loop(0, 20, body, (c1, _SPAN))

    @pl.when(jnp.logical_not(ok))
    def _full_search():
        neg = red(m < 0)
        res0 = jnp.where(neg >= k, _INT_MIN, np.int32(0))
        full, _ = jax.lax.fori_loop(0, 31, body, (res0, np.int32(1 << 30)))
        full_ref[...] = full

    m_lo = jnp.where(ok, short, full_ref[...])
    prev_ref[...] = m_lo
    cnt_le = red(m <= m_lo)
    m_hi_next = jnp.min(jnp.where(m > m_lo, m, _INT_MAX), axis=(1, 2),
                        keepdims=True)
    m_hi = jnp.where(cnt_le >= k + 1, m_lo, m_hi_next)
    f_lo = jax.lax.bitcast_convert_type(_sortable(m_lo), jnp.float32)
    f_hi = jax.lax.bitcast_convert_type(_sortable(m_hi), jnp.float32)
    return f_lo * _W_LO + f_hi * _W_HI


def _dot(x, y):
    return jax.lax.dot_general(x, y, (((1,), (0,)), ((), ())),
                               preferred_element_type=jnp.float32)


BPS = 16
HBPS = 16


def _h3_body(v_ref, wiT_ref, bi_ref, h3_ref):
    h3_ref[...] = (_dot(v_ref[...].reshape(HBPS * N, N), wiT_ref[...])
                   + bi_ref[...]).reshape(HBPS, N, HID)


def _stage1_body(h3_ref, a_ref, eps_ref,
                 A1_ref, c1_ref, A2_ref, c2_ref,
                 Ae_ref, ce_ref, WaT_ref, ba_ref,
                 hr_ref, ga_ref, prev_ref, full_ref):
    aa = a_ref[...]
    pct = _percentile_blocks(aa, prev_ref, full_ref,
                             pl.program_id(0))
    h3 = h3_ref[...]
    mask = (aa > pct).astype(jnp.float32)
    M = jax.lax.dot_general(mask, h3, (((2,), (1,)), ((0,), (0,))),
                            preferred_element_type=jnp.float32)
    for l in range(NLAYERS):
        x = (M + eps_ref[l] * h3).reshape(BPS * N, HID)
        x = jax.nn.relu(_dot(x, A1_ref[l]) + c1_ref[l])
        x2 = jax.nn.relu(_dot(x, A2_ref[l]) + c2_ref[l]).reshape(BPS, N, HID)
        xr = jnp.mean(x2, axis=1)
        pe = _dot(xr, Ae_ref[l]) + ce_ref[l]
        xe = pe * (jax.lax.erf(pe / np.float32(np.sqrt(2.0))) + 1) / 2
        ga = jax.nn.sigmoid(_dot(xe, WaT_ref[l]) + ba_ref[l])
        h_read = jnp.sum(x2 * ga[:, :, None], axis=1) * np.float32(1.0 / N)
        ga_ref[:, l, :] = ga
        hr_ref[:, l, :] = h_read


def _ln(x, g, b):
    m = x.mean(-1, keepdims=True)
    va = ((x - m) ** 2).mean(-1, keepdims=True)
    return g * (x - m) / jnp.sqrt(va + 1e-5) + b


def _bdot(x, y):
    return jax.lax.dot_general(x, y, (((2,), (1,)), ((0,), (0,))),
                               preferred_element_type=jnp.float32)


def _stage2_body(hr_ref, bias_ref,
                 WqT_ref, bq_ref, WkT_ref, bk_ref, WvT_ref, bv_ref,
                 WoT_ref, bo_ref, ln1g_ref, ln1b_ref,
                 W1T_ref, b1_ref, W2T_ref, b2_ref, ln2g_ref, ln2b_ref,
                 WLT_ref, bL_ref,
                 logit_ref, ta_ref, fG_ref, fT_ref, fL_ref):
    LB = NLAYERS * B
    hr4 = hr_ref[...]
    fG_ref[...] = jnp.mean(hr4, axis=2)
    hr2 = hr4.reshape(NLAYERS, B * T, HID)
    q = (_bdot(hr2, WqT_ref[...]) + bq_ref[...]).reshape(LB * T, HID)
    k = (_bdot(hr2, WkT_ref[...]) + bk_ref[...]).reshape(LB * T, HID)
    vv = (_bdot(hr2, WvT_ref[...]) + bv_ref[...]).reshape(LB * T, HID)
    sc = _dot(q, k.T) / np.float32(np.sqrt(float(HID))) + bias_ref[...]
    w = jax.nn.softmax(sc, axis=-1)
    for lb in range(LB):
        ta_ref[lb // B, lb % B, :, :] = \
            w[lb * T:(lb + 1) * T, lb * T:(lb + 1) * T]
    o = _dot(w, vv).reshape(NLAYERS, B * T, HID)
    o = _bdot(o, WoT_ref[...]) + bo_ref[...]
    xa = _ln(o, ln1g_ref[...], ln1b_ref[...])
    x2 = _bdot(jax.nn.relu(_bdot(xa, W1T_ref[...]) + b1_ref[...]),
               W2T_ref[...]) + b2_ref[...]
    xa = _ln(xa + x2, ln2g_ref[...], ln2b_ref[...])
    featT = jnp.sum(xa.reshape(NLAYERS, B, T, HID), axis=2)
    featL = _bdot(featT, WLT_ref[...]) + bL_ref[...]
    fT_ref[...] = featT
    fL_ref[...] = featL
    logit_ref[...] = jnp.sum(featL, axis=0)


def _bn_fold(p):
    s = p['g'] / jnp.sqrt(p['v'] + 1e-5)
    return s, p['b'] - p['m'] * s


@jax.jit
def kernel(v, a, t, sampling_endpoints, params):
    del t, sampling_endpoints
    layers = params['layers']
    v3 = v.reshape(B * T, N, N)
    a3 = a.reshape(B * T, N, N)

    wiT = params['init_W'].T
    bi = params['init_b'].reshape(1, HID)
    eps = jnp.stack([p['eps'] for p in layers])

    def fold_lin(W, bvec, bn):
        s, sh = _bn_fold(bn)
        return W.T * s[None, :], (bvec * s + sh).reshape(1, -1)

    A1, c1, A2, c2, Ae, ce, WaT, ba = [], [], [], [], [], [], [], []
    for p in layers:
        w, c = fold_lin(p['g_W1'], p['g_b1'], p['g_bn1']); A1.append(w); c1.append(c)
        w, c = fold_lin(p['g_W2'], p['g_b2'], p['g_bn2']); A2.append(w); c2.append(c)
        w, c = fold_lin(p['s_We'], p['s_be'], p['s_bn']); Ae.append(w); ce.append(c)
        WaT.append(p['s_Wa'].T); ba.append(p['s_ba'].reshape(1, N))
    st = jnp.stack

    def full(shape):
        return pl.BlockSpec(shape, lambda i: (0,) * len(shape))

    def blk(shape):
        return pl.BlockSpec(shape, lambda i: (i,) + (0,) * (len(shape) - 1))

    h3_all = pl.pallas_call(
        _h3_body,
        grid=(B * T // HBPS,),
        in_specs=[blk((HBPS, N, N)), full((N, HID)), full((1, HID))],
        out_specs=blk((HBPS, N, HID)),
        out_shape=jax.ShapeDtypeStruct((B * T, N, HID), jnp.float32),
    )(v3, wiT, bi)

    hr_all, ga_all = pl.pallas_call(
        _stage1_body,
        grid=(B * T // BPS,),
        in_specs=[
            blk((BPS, N, HID)), blk((BPS, N, N)),
            full((NLAYERS, 1, 1)),
            full((NLAYERS, HID, HID)), full((NLAYERS, 1, HID)),
            full((NLAYERS, HID, HID)), full((NLAYERS, 1, HID)),
            full((NLAYERS, HID, HID)), full((NLAYERS, 1, HID)),
            full((NLAYERS, HID, N)), full((NLAYERS, 1, N)),
        ],
        out_specs=[blk((BPS, NLAYERS, HID)), blk((BPS, NLAYERS, N))],
        out_shape=[
            jax.ShapeDtypeStruct((B * T, NLAYERS, HID), jnp.float32),
            jax.ShapeDtypeStruct((B * T, NLAYERS, N), jnp.float32),
        ],
        scratch_shapes=[pltpu.VMEM((BPS, 1, 1), jnp.int32),
                        pltpu.VMEM((BPS, 1, 1), jnp.int32)],
    )(h3_all, a3, eps, st(A1), st(c1), st(A2), st(c2),
      st(Ae), st(ce), st(WaT), st(ba))

    hr = hr_all.reshape(B, T, NLAYERS, HID).transpose(2, 0, 1, 3)
    node_att = ga_all.reshape(B, T, NLAYERS, N).transpose(0, 2, 1, 3)

    Wq, bq, Wk, bk, Wv, bv = [], [], [], [], [], []
    Wo, bo, g1, b1g, W1m, b1m, W2m, b2m, g2, b2g, WL, bL = ([] for _ in range(12))
    for p in layers:
        q3, k3, v3s = jnp.split(p['t_Win'], 3, axis=0)
        q3b, k3b, v3b = jnp.split(p['t_bin'], 3)
        Wq.append(q3.T); Wk.append(k3.T); Wv.append(v3s.T)
        bq.append(q3b.reshape(1, HID)); bk.append(k3b.reshape(1, HID))
        bv.append(v3b.reshape(1, HID))
        Wo.append(p['t_Wo'].T); bo.append(p['t_bo'].reshape(1, HID))
        g1.append(p['t_ln1g'].reshape(1, HID)); b1g.append(p['t_ln1b'].reshape(1, HID))
        W1m.append(p['t_W1'].T); b1m.append(p['t_b1'].reshape(1, 2 * HID))
        W2m.append(p['t_W2'].T); b2m.append(p['t_b2'].reshape(1, HID))
        g2.append(p['t_ln2g'].reshape(1, HID)); b2g.append(p['t_ln2b'].reshape(1, HID))
        WL.append(p['L_W'].T); bL.append(p['L_b'].reshape(1, NCLS))

    s2_ins = [st(Wq), st(bq), st(Wk), st(bk), st(Wv), st(bv),
              st(Wo), st(bo), st(g1), st(b1g),
              st(W1m), st(b1m), st(W2m), st(b2m), st(g2), st(b2g),
              st(WL), st(bL)]

    lbt = NLAYERS * B * T
    blkid = jnp.arange(lbt, dtype=jnp.int32) // T
    bias = jnp.where(blkid[:, None] == blkid[None, :],
                     jnp.float32(0), -jnp.inf)

    outs = pl.pallas_call(
        _stage2_body,
        grid=(1,),
        in_specs=[full((NLAYERS, B, T, HID)), full((lbt, lbt))]
                 + [full(x.shape) for x in s2_ins],
        out_specs=[full((B, NCLS)), full((NLAYERS, B, T, T)),
                   full((NLAYERS, B, HID)), full((NLAYERS, B, HID)),
                   full((NLAYERS, B, NCLS))],
        out_shape=[
            jax.ShapeDtypeStruct((B, NCLS), jnp.float32),
            jax.ShapeDtypeStruct((NLAYERS, B, T, T), jnp.float32),
            jax.ShapeDtypeStruct((NLAYERS, B, HID), jnp.float32),
            jax.ShapeDtypeStruct((NLAYERS, B, HID), jnp.float32),
            jax.ShapeDtypeStruct((NLAYERS, B, NCLS), jnp.float32),
        ],
    )(hr, bias, *s2_ins)

    logit, ta, fG, fT, fL = outs
    return (logit,
            node_att,
            ta.transpose(1, 0, 2, 3),
            fG.transpose(1, 0, 2),
            fT.transpose(1, 0, 2),
            fL.transpose(1, 0, 2))
